# speedup vs baseline: 14.7898x; 14.7898x over previous
import jax
import jax.numpy as jnp
import numpy as np
from jax.experimental import pallas as pl
from jax.experimental.pallas import tpu as pltpu

_EPS = 1e-06
_S = 0.025
_ALPHA = 0.98
_DELTA = 2.0
_R = 0.5

_TB = 256


def _pcen_block_kernel(x_ref, l_ref, o_ref, carry_ref):
    t = pl.program_id(1)

    @pl.when(t == 0)
    def _():
        carry_ref[...] = jnp.zeros_like(carry_ref)

    x = x_ref[0]
    m = jax.lax.dot_general(
        x,
        l_ref[...],
        (((1,), (0,)), ((), ())),
        preferred_element_type=jnp.float32,
        precision=jax.lax.Precision.HIGHEST,
    )
    decay = l_ref[0:1, :] * ((1.0 - _S) / _S)
    m = m + carry_ref[...] * decay
    carry_ref[...] = m[:, _TB - 1 : _TB]
    o_ref[0] = (
        jnp.sqrt(x * jnp.exp(-_ALPHA * jnp.log(m + _EPS)) + _DELTA)
        - np.float32(np.sqrt(_DELTA))
    )


@jax.jit
def kernel(x):
    B, F, T = x.shape
    nt = T // _TB
    j = np.arange(_TB)
    diff = j[None, :] - j[:, None]
    L = np.where(diff >= 0, _S * (1.0 - _S) ** diff, 0.0)
    L = jnp.asarray(L, dtype=jnp.float32)

    return pl.pallas_call(
        _pcen_block_kernel,
        grid=(B, nt),
        in_specs=[
            pl.BlockSpec((1, F, _TB), lambda b, t: (b, 0, t)),
            pl.BlockSpec((_TB, _TB), lambda b, t: (0, 0)),
        ],
        out_specs=pl.BlockSpec((1, F, _TB), lambda b, t: (b, 0, t)),
        out_shape=jax.ShapeDtypeStruct((B, F, T), jnp.float32),
        scratch_shapes=[pltpu.VMEM((F, 1), jnp.float32)],
        compiler_params=pltpu.CompilerParams(
            dimension_semantics=("parallel", "arbitrary")
        ),
    )(x, L)


# speedup vs baseline: 36.1041x; 2.4412x over previous
import jax
import jax.numpy as jnp
import numpy as np
from jax.experimental import pallas as pl
from jax.experimental.pallas import tpu as pltpu

_EPS = 1e-06
_S = 0.025
_ALPHA = 0.98
_DELTA = 2.0
_R = 0.5

_TC = 256


def _pcen_kernel(x_ref, l_ref, o_ref):
    F = x_ref.shape[1]
    T = x_ref.shape[2]
    nchunks = T // _TC
    decay = l_ref[0:1, :] * ((1.0 - _S) / _S)
    sqrt_delta = np.float32(np.sqrt(_DELTA))

    def body(c, carry):
        x = x_ref[0, :, pl.ds(c * _TC, _TC)]
        m = jax.lax.dot_general(
            x,
            l_ref[...],
            (((1,), (0,)), ((), ())),
            preferred_element_type=jnp.float32,
        )
        m = m + carry * decay
        o_ref[0, :, pl.ds(c * _TC, _TC)] = (
            jnp.sqrt(x * jnp.exp(-_ALPHA * jnp.log(m + _EPS)) + _DELTA)
            - sqrt_delta
        )
        return m[:, _TC - 1 : _TC]

    jax.lax.fori_loop(0, nchunks, body, jnp.zeros((F, 1), jnp.float32))


@jax.jit
def kernel(x):
    B, F, T = x.shape
    j = np.arange(_TC)
    diff = j[None, :] - j[:, None]
    L = np.where(diff >= 0, _S * (1.0 - _S) ** diff, 0.0)
    L = jnp.asarray(L, dtype=jnp.float32)

    return pl.pallas_call(
        _pcen_kernel,
        grid=(B,),
        in_specs=[
            pl.BlockSpec((1, F, T), lambda b: (b, 0, 0)),
            pl.BlockSpec((_TC, _TC), lambda b: (0, 0)),
        ],
        out_specs=pl.BlockSpec((1, F, T), lambda b: (b, 0, 0)),
        out_shape=jax.ShapeDtypeStruct((B, F, T), jnp.float32),
        compiler_params=pltpu.CompilerParams(
            dimension_semantics=("parallel",)
        ),
    )(x, L)


# speedup vs baseline: 45.9207x; 1.2719x over previous
import jax
import jax.numpy as jnp
import numpy as np
from jax.experimental import pallas as pl
from jax.experimental.pallas import tpu as pltpu

_EPS = 1e-06
_S = 0.025
_ALPHA = 0.98
_DELTA = 2.0
_R = 0.5

_TC = 256


def _pcen_kernel(x_ref, l_ref, o_ref):
    F = x_ref.shape[1]
    T = x_ref.shape[2]
    lmat = l_ref[...]
    decay = lmat[0:1, :] * ((1.0 - _S) / _S)
    lmat_bf = lmat.astype(jnp.bfloat16)
    sqrt_delta = np.float32(np.sqrt(_DELTA))

    def ema_chunk(c):
        x = x_ref[0, :, pl.ds(c * _TC, _TC)]
        y = jax.lax.dot_general(
            x.astype(jnp.bfloat16),
            lmat_bf,
            (((1,), (0,)), ((), ())),
            preferred_element_type=jnp.float32,
        )
        return x, y

    def compress(c, x, m):
        o_ref[0, :, pl.ds(c * _TC, _TC)] = (
            jnp.sqrt(x * jnp.exp2(-_ALPHA * jnp.log2(m + _EPS)) + _DELTA)
            - sqrt_delta
        )

    def body(i, carry):
        c0 = 2 * i
        x0, y0 = ema_chunk(c0)
        x1, y1 = ema_chunk(c0 + 1)
        m0 = y0 + carry * decay
        carry = m0[:, _TC - 1 : _TC]
        m1 = y1 + carry * decay
        compress(c0, x0, m0)
        compress(c0 + 1, x1, m1)
        return m1[:, _TC - 1 : _TC]

    jax.lax.fori_loop(0, T // (2 * _TC), body, jnp.zeros((F, 1), jnp.float32))


@jax.jit
def kernel(x):
    B, F, T = x.shape
    j = np.arange(_TC)
    diff = j[None, :] - j[:, None]
    L = np.where(diff >= 0, _S * (1.0 - _S) ** diff, 0.0)
    L = jnp.asarray(L, dtype=jnp.float32)

    return pl.pallas_call(
        _pcen_kernel,
        grid=(B,),
        in_specs=[
            pl.BlockSpec((1, F, T), lambda b: (b, 0, 0)),
            pl.BlockSpec((_TC, _TC), lambda b: (0, 0)),
        ],
        out_specs=pl.BlockSpec((1, F, T), lambda b: (b, 0, 0)),
        out_shape=jax.ShapeDtypeStruct((B, F, T), jnp.float32),
        compiler_params=pltpu.CompilerParams(
            dimension_semantics=("parallel",)
        ),
    )(x, L)


# speedup vs baseline: 64.1236x; 1.3964x over previous
import jax
import jax.numpy as jnp
import numpy as np
from jax.experimental import pallas as pl
from jax.experimental.pallas import tpu as pltpu

_EPS = 1e-06
_S = 0.025
_ALPHA = 0.98
_DELTA = 2.0
_R = 0.5

_TC = 256


def _pcen_kernel(x_ref, l_ref, o_ref):
    F = x_ref.shape[1]
    T = x_ref.shape[2]
    lmat = l_ref[...]
    decay = lmat[0:1, :] * ((1.0 - _S) / _S)
    lmat_bf = lmat.astype(jnp.bfloat16)
    sqrt_delta = np.float32(np.sqrt(_DELTA))

    def ema_chunk(c):
        x = x_ref[0, :, pl.ds(c * _TC, _TC)]
        y = jax.lax.dot_general(
            x.astype(jnp.bfloat16),
            lmat_bf,
            (((1,), (0,)), ((), ())),
            preferred_element_type=jnp.float32,
        )
        return x, y

    def compress(c, x, m):
        u = x * jnp.exp2(-_ALPHA * jnp.log2(m + _EPS)) + _DELTA
        o_ref[0, :, pl.ds(c * _TC, _TC)] = u * jax.lax.rsqrt(u) - sqrt_delta

    carry = jnp.zeros((F, 1), jnp.float32)
    for c in range(T // _TC):
        x, y = ema_chunk(c)
        m = y + carry * decay
        carry = m[:, _TC - 1 : _TC]
        compress(c, x, m)


@jax.jit
def kernel(x):
    B, F, T = x.shape
    j = np.arange(_TC)
    diff = j[None, :] - j[:, None]
    L = np.where(diff >= 0, _S * (1.0 - _S) ** diff, 0.0)
    L = jnp.asarray(L, dtype=jnp.float32)

    return pl.pallas_call(
        _pcen_kernel,
        grid=(B,),
        in_specs=[
            pl.BlockSpec((1, F, T), lambda b: (b, 0, 0)),
            pl.BlockSpec((_TC, _TC), lambda b: (0, 0)),
        ],
        out_specs=pl.BlockSpec((1, F, T), lambda b: (b, 0, 0)),
        out_shape=jax.ShapeDtypeStruct((B, F, T), jnp.float32),
        compiler_params=pltpu.CompilerParams(
            dimension_semantics=("parallel",)
        ),
    )(x, L)
